# Initial kernel scaffold; baseline (speedup 1.0000x reference)
import jax
import jax.numpy as jnp
from jax.experimental import pallas as pl


def kernel(x):
    raise NotImplementedError("write your pallas kernel here")


# speedup vs baseline: 2.0990x; 2.0990x over previous
import functools

import numpy as np
import jax
import jax.numpy as jnp
from jax.experimental import pallas as pl

_RATIO_INTERVAL = (0.05, 0.1)
_NOISE_COUNT_INTERVAL = (1, 5)
_H_MARGINS = (5, 5)
_W_MARGINS = (5, 5)

_rk = jax.random.key(42)
_k1, _k2, _k3, _k4, _k5, _k6 = jax.random.split(_rk, 6)
with jax.ensure_compile_time_eval():
    _NOISE_COUNT = int(jax.random.randint(
        _k2, (1,), _NOISE_COUNT_INTERVAL[0], _NOISE_COUNT_INTERVAL[1])[0])
    _H_CHOICE = int(jax.random.randint(_k3, (1,), 0, 2)[0])
    _W_CHOICE = int(jax.random.randint(_k4, (1,), 0, 2)[0])
    _RATIO = float(
        jax.random.uniform(_k1, (1,), dtype=jnp.float32)[0]
        * (_RATIO_INTERVAL[1] - _RATIO_INTERVAL[0]) + _RATIO_INTERVAL[0])


@functools.lru_cache(maxsize=None)
def _noise_points(h, w):
    with jax.ensure_compile_time_eval():
        h_int = [(0, _H_MARGINS[0]), (h - _H_MARGINS[1], h)][_H_CHOICE]
        w_int = [(0, _W_MARGINS[0]), (w - _W_MARGINS[1], w)][_W_CHOICE]
        h_idx = np.asarray(jax.random.randint(
            _k5, (_NOISE_COUNT,), h_int[0], h_int[1]))
        w_idx = np.asarray(jax.random.randint(
            _k6, (_NOISE_COUNT,), w_int[0], w_int[1]))
    mult = {}
    for hv, wv in zip(h_idx, w_idx):
        f = int(hv) * w + int(wv)
        mult[f] = mult.get(f, 0) + 1
    return tuple(sorted(mult.items()))


def _body(points, x_ref, o_ref):
    v = x_ref[...]
    m = jnp.max(v, axis=1, keepdims=True) * jnp.float32(_RATIO)
    o_ref[...] = v
    for col, k in points:
        o_ref[:, col:col + 1] = v[:, col:col + 1] + jnp.float32(k) * m


@jax.jit
def kernel(x):
    b, c, h, w = x.shape
    n = b * c
    sp = h * w
    points = _noise_points(h, w)

    cb = 16
    while n % cb != 0:
        cb //= 2

    x2 = x.reshape(n, sp)
    y2 = pl.pallas_call(
        functools.partial(_body, points),
        grid=(n // cb,),
        in_specs=[pl.BlockSpec((cb, sp), lambda i: (i, 0))],
        out_specs=pl.BlockSpec((cb, sp), lambda i: (i, 0)),
        out_shape=jax.ShapeDtypeStruct((n, sp), x.dtype),
    )(x2)
    return y2.reshape(b, c, h, w)
